# baseline (speedup 1.0000x reference)
import jax
import jax.numpy as jnp
from jax import lax
from jax.experimental import pallas as pl
from jax.experimental.pallas import tpu as pltpu
from jax.experimental.pallas import tpu_sc as plsc

DIM = 128
SEQ = 50
SEQ_PAD = 56
NC = 2
NS = 16
NW = NC * NS
BATCH = 4096
BPW = BATCH // NW
NBUF = 8
NGROUP = BPW // NBUF


def _body(x_hbm, tbl_hbm, out_hbm, idx_v, rows_v, gsem, osem):
    wid = lax.axis_index("s") * NC + lax.axis_index("c")
    pltpu.sync_copy(x_hbm.at[pl.ds(wid * BPW, BPW)], idx_v)

    def start_gather(b, buf):
        pltpu.async_copy(
            tbl_hbm.at[idx_v.at[b, pl.ds(0, SEQ)]],
            rows_v.at[buf, pl.ds(0, SEQ)], gsem.at[buf])

    def wait_gather(buf):
        pltpu.make_async_copy(
            tbl_hbm.at[idx_v.at[0, pl.ds(0, SEQ)]],
            rows_v.at[buf, pl.ds(0, SEQ)], gsem.at[buf]).wait()

    def start_out(b, buf):
        pltpu.async_copy(
            rows_v.at[buf, pl.ds(0, SEQ)], out_hbm.at[wid * BPW + b],
            osem.at[buf])

    def wait_out(buf):
        pltpu.make_async_copy(
            rows_v.at[buf, pl.ds(0, SEQ)], out_hbm.at[0], osem.at[buf]).wait()

    for buf in range(NBUF):
        start_gather(buf, buf)

    def group(g, carry):
        for buf in range(NBUF):
            wait_gather(buf)
            start_out(g * NBUF + buf, buf)
        for buf in range(NBUF):
            wait_out(buf)

            @pl.when(g + 1 < NGROUP)
            def _():
                start_gather((g + 1) * NBUF + buf, buf)

        return carry

    lax.fori_loop(0, NGROUP, group, 0)


@jax.jit
def _run(x, emb_weight):
    mesh = plsc.VectorSubcoreMesh(core_axis_name="c", subcore_axis_name="s")
    k = pl.kernel(
        _body,
        out_type=jax.ShapeDtypeStruct((BATCH, SEQ, DIM), jnp.float32),
        mesh=mesh,
        scratch_types=[
            pltpu.VMEM((BPW, SEQ), jnp.int32),
            pltpu.VMEM((NBUF, SEQ_PAD, DIM), jnp.float32),
            pltpu.SemaphoreType.DMA((NBUF,)),
            pltpu.SemaphoreType.DMA((NBUF,)),
        ],
    )
    return k(x, emb_weight)


def kernel(x, emb_weight):
    return _run(x.astype(jnp.int32), emb_weight)
